# baseline (speedup 1.0000x reference)
import functools

import jax
import jax.numpy as jnp
from jax import lax
from jax.experimental import pallas as pl
from jax.experimental.pallas import tpu as pltpu
from jax.experimental.pallas import tpu_sc as plsc

N = 100000
E = 3200000
D_IN = 128
H = 16
G = 64

NPAD = 100352
NTILE = NPAD // 16
ZR = 784

EPW = E // 32
SB = 800
OUTER = EPW // SB

DSB = 4000
DOUTER = EPW // DSB

R = 4000
GRID = N // R

_mesh = plsc.VectorSubcoreMesh(core_axis_name="c", subcore_axis_name="s")
_sc_params = pltpu.CompilerParams(use_tc_tiling_on_sc=False)


@functools.partial(
    pl.kernel,
    out_type=jax.ShapeDtypeStruct((2, NPAD), jnp.float32),
    mesh=_mesh,
    compiler_params=_sc_params,
    scratch_types=[
        pltpu.VMEM((2, DSB), jnp.int32),
        pltpu.VMEM((2, DSB), jnp.float32),
        pltpu.VMEM((NTILE,), jnp.float32),
        pltpu.VMEM_SHARED((NPAD,), jnp.float32),
        pltpu.SemaphoreType.DMA((2,)),
        pltpu.SemaphoreType.DMA((2,)),
    ],
)
def _sc_deg(edge_hbm, ew_hbm, out_hbm, dstb, ewb, zb, acc, esem, ssem):
    cid = lax.axis_index("c")
    sid = lax.axis_index("s")
    wid = cid * 16 + sid
    wbase = wid * EPW

    def zloop(i, _):
        zb[pl.ds(i * 16, 16)] = jnp.zeros((16,), jnp.float32)
        return 0

    lax.fori_loop(0, NTILE // 16, zloop, 0)
    pltpu.sync_copy(zb, acc.at[pl.ds(sid * NTILE, NTILE)])
    plsc.subcore_barrier()

    def stage(kc, b):
        e0 = wbase + kc * DSB
        pltpu.async_copy(edge_hbm.at[1, pl.ds(e0, DSB)], dstb.at[b],
                         esem.at[b])
        pltpu.async_copy(ew_hbm.at[pl.ds(e0, DSB)], ewb.at[b], esem.at[b])

    def unstage(kc, b):
        e0 = wbase + kc * DSB
        pltpu.make_async_copy(edge_hbm.at[1, pl.ds(e0, DSB)], dstb.at[b],
                              esem.at[b]).wait()
        pltpu.make_async_copy(ew_hbm.at[pl.ds(e0, DSB)], ewb.at[b],
                              esem.at[b]).wait()

    stage(0, 0)

    def step(kc, _):
        b = kc % 2
        unstage(kc, b)

        @pl.when(kc < DOUTER - 1)
        def _():
            @pl.when(kc >= 1)
            def _():
                pltpu.make_async_copy(ewb.at[1 - b],
                                      acc.at[dstb.at[1 - b]],
                                      ssem.at[1 - b]).wait()
            stage(kc + 1, 1 - b)

        pltpu.async_copy(ewb.at[b], acc.at[dstb.at[b]], ssem.at[b],
                         add=True)
        return 0

    lax.fori_loop(0, DOUTER, step, 0)
    pltpu.make_async_copy(ewb.at[0], acc.at[dstb.at[0]], ssem.at[0]).wait()
    pltpu.make_async_copy(ewb.at[1], acc.at[dstb.at[1]], ssem.at[1]).wait()
    plsc.subcore_barrier()
    sl = pl.ds(sid * NTILE, NTILE)
    pltpu.sync_copy(acc.at[sl], out_hbm.at[cid, sl])


@functools.partial(
    pl.kernel,
    out_type=jax.ShapeDtypeStruct((2, NPAD, H), jnp.float32),
    mesh=_mesh,
    compiler_params=_sc_params,
    scratch_types=[
        pltpu.VMEM((2, SB), jnp.int32),
        pltpu.VMEM((2, SB), jnp.int32),
        pltpu.VMEM((2, SB), jnp.float32),
        pltpu.VMEM((2, SB, H), jnp.float32),
        pltpu.VMEM_SHARED((NPAD, H), jnp.float32),
        pltpu.SemaphoreType.DMA((2,)),
        pltpu.SemaphoreType.DMA((2,)),
        pltpu.SemaphoreType.DMA((2,)),
    ],
)
def _sc_agg(edge_hbm, ew_hbm, hs_hbm, out_hbm,
            srcb, dstb, ewb, rows, acc, esem, gsem, ssem):
    cid = lax.axis_index("c")
    sid = lax.axis_index("s")
    wid = cid * 16 + sid
    wbase = wid * EPW

    def zloop(i, _):
        rows[0, i] = jnp.zeros((H,), jnp.float32)
        return 0

    lax.fori_loop(0, ZR, zloop, 0)
    base = sid * NTILE
    for t in range(NTILE // ZR):
        pltpu.sync_copy(rows.at[0, pl.ds(0, ZR)],
                        acc.at[pl.ds(base + t * ZR, ZR)])
    plsc.subcore_barrier()

    def stage(kc, b):
        e0 = wbase + kc * SB
        pltpu.async_copy(edge_hbm.at[0, pl.ds(e0, SB)], srcb.at[b],
                         esem.at[b])
        pltpu.async_copy(edge_hbm.at[1, pl.ds(e0, SB)], dstb.at[b],
                         esem.at[b])
        pltpu.async_copy(ew_hbm.at[pl.ds(e0, SB)], ewb.at[b], esem.at[b])

    def unstage(kc, b):
        e0 = wbase + kc * SB
        pltpu.make_async_copy(edge_hbm.at[0, pl.ds(e0, SB)], srcb.at[b],
                              esem.at[b]).wait()
        pltpu.make_async_copy(edge_hbm.at[1, pl.ds(e0, SB)], dstb.at[b],
                              esem.at[b]).wait()
        pltpu.make_async_copy(ew_hbm.at[pl.ds(e0, SB)], ewb.at[b],
                              esem.at[b]).wait()

    stage(0, 0)

    def step(kc, _):
        b = kc % 2
        unstage(kc, b)
        pltpu.async_copy(hs_hbm.at[srcb.at[b]], rows.at[b], gsem.at[b])

        @pl.when(kc < OUTER - 1)
        def _():
            @pl.when(kc >= 1)
            def _():
                pltpu.make_async_copy(rows.at[1 - b],
                                      acc.at[dstb.at[1 - b]],
                                      ssem.at[1 - b]).wait()
            stage(kc + 1, 1 - b)

        pltpu.make_async_copy(hs_hbm.at[srcb.at[b]], rows.at[b],
                              gsem.at[b]).wait()

        def scale(g, _):
            e0 = g * 16
            ew16 = ewb[b, pl.ds(e0, 16)]
            for l in range(16):
                rows[b, e0 + l] = rows[b, e0 + l] * ew16[l]
            return 0

        lax.fori_loop(0, SB // 16, scale, 0)
        pltpu.async_copy(rows.at[b], acc.at[dstb.at[b]], ssem.at[b],
                         add=True)
        return 0

    lax.fori_loop(0, OUTER, step, 0)
    pltpu.make_async_copy(rows.at[0], acc.at[dstb.at[0]], ssem.at[0]).wait()
    pltpu.make_async_copy(rows.at[1], acc.at[dstb.at[1]], ssem.at[1]).wait()
    plsc.subcore_barrier()
    sl = pl.ds(base, NTILE)
    pltpu.sync_copy(acc.at[sl], out_hbm.at[cid, sl])


def _mm_body(x_ref, w_ref, h_ref):
    h_ref[...] = jnp.dot(x_ref[...], w_ref[...],
                         preferred_element_type=jnp.float32)


_tc_mm = pl.pallas_call(
    _mm_body,
    grid=(GRID,),
    in_specs=[
        pl.BlockSpec((R, D_IN), lambda i: (i, 0)),
        pl.BlockSpec((D_IN, H), lambda i: (0, 0)),
    ],
    out_specs=pl.BlockSpec((R, H), lambda i: (i, 0)),
    out_shape=jax.ShapeDtypeStruct((N, H), jnp.float32),
)


def _scale_body(degp_ref, h_ref, hs_ref, dinv_ref):
    deg = 1.0 + degp_ref[0, 0] + degp_ref[1, 0]
    dinv_row = jnp.where(deg > 0, lax.rsqrt(deg), 0.0)
    dinv = jnp.transpose(dinv_row)
    hs_ref[...] = dinv * h_ref[...]
    dinv_ref[...] = jnp.reshape(dinv_row, (1, 1, R))


_tc_scale = pl.pallas_call(
    _scale_body,
    grid=(GRID,),
    in_specs=[
        pl.BlockSpec((2, 1, 1, R), lambda i: (0, i, 0, 0)),
        pl.BlockSpec((R, H), lambda i: (i, 0)),
    ],
    out_specs=[
        pl.BlockSpec((R, H), lambda i: (i, 0)),
        pl.BlockSpec((1, 1, R), lambda i: (i, 0, 0)),
    ],
    out_shape=[
        jax.ShapeDtypeStruct((N, H), jnp.float32),
        jax.ShapeDtypeStruct((GRID, 1, R), jnp.float32),
    ],
)


def _mid_body(accp_ref, h1_ref, dinv_ref, b1_ref, w2_ref, h2_ref, hs2_ref):
    dinv = jnp.transpose(dinv_ref[0])
    agg = accp_ref[0] + accp_ref[1]
    z = jnp.maximum(b1_ref[...] + dinv * agg + dinv * dinv * h1_ref[...], 0.0)
    h2 = jnp.dot(z, w2_ref[...], preferred_element_type=jnp.float32)
    h2_ref[...] = h2
    hs2_ref[...] = dinv * h2


_tc_mid = pl.pallas_call(
    _mid_body,
    grid=(GRID,),
    in_specs=[
        pl.BlockSpec((2, R, H), lambda i: (0, i, 0)),
        pl.BlockSpec((R, H), lambda i: (i, 0)),
        pl.BlockSpec((1, 1, R), lambda i: (i, 0, 0)),
        pl.BlockSpec((1, H), lambda i: (0, 0)),
        pl.BlockSpec((H, H), lambda i: (0, 0)),
    ],
    out_specs=[
        pl.BlockSpec((R, H), lambda i: (i, 0)),
        pl.BlockSpec((R, H), lambda i: (i, 0)),
    ],
    out_shape=[
        jax.ShapeDtypeStruct((N, H), jnp.float32),
        jax.ShapeDtypeStruct((N, H), jnp.float32),
    ],
)


def _final_body(accp_ref, h2_ref, dinv_ref, b2_ref, batch_ref, wfc_ref,
                bfc_ref, out_ref, acc, cnt):
    i = pl.program_id(0)

    @pl.when(i == 0)
    def _():
        acc[...] = jnp.zeros((G, H), jnp.float32)
        cnt[...] = jnp.zeros((G, 1), jnp.float32)

    dinv = jnp.transpose(dinv_ref[0])
    agg = accp_ref[0] + accp_ref[1]
    z = jnp.maximum(b2_ref[...] + dinv * agg + dinv * dinv * h2_ref[...], 0.0)
    b = batch_ref[0]
    oht = (lax.broadcasted_iota(jnp.int32, (G, R), 0) == b).astype(jnp.float32)
    acc[...] += lax.dot_general(oht, z, (((1,), (0,)), ((), ())),
                                preferred_element_type=jnp.float32)
    cnt[...] += lax.dot_general(oht, jnp.ones((R, 1), jnp.float32),
                                (((1,), (0,)), ((), ())),
                                preferred_element_type=jnp.float32)

    @pl.when(i == GRID - 1)
    def _():
        pooled = acc[...] / jnp.maximum(cnt[...], 1.0)
        logits = jnp.dot(pooled, wfc_ref[...],
                         preferred_element_type=jnp.float32) + bfc_ref[...]
        m = jnp.max(logits, axis=1, keepdims=True)
        lse = m + jnp.log(jnp.sum(jnp.exp(logits - m), axis=1, keepdims=True))
        out_ref[...] = logits - lse


_tc_final = pl.pallas_call(
    _final_body,
    grid=(GRID,),
    in_specs=[
        pl.BlockSpec((2, R, H), lambda i: (0, i, 0)),
        pl.BlockSpec((R, H), lambda i: (i, 0)),
        pl.BlockSpec((1, 1, R), lambda i: (i, 0, 0)),
        pl.BlockSpec((1, H), lambda i: (0, 0)),
        pl.BlockSpec((1, 1, R), lambda i: (i, 0, 0)),
        pl.BlockSpec((H, 2), lambda i: (0, 0)),
        pl.BlockSpec((1, 2), lambda i: (0, 0)),
    ],
    out_specs=pl.BlockSpec((G, 2), lambda i: (0, 0)),
    out_shape=jax.ShapeDtypeStruct((G, 2), jnp.float32),
    scratch_shapes=[
        pltpu.VMEM((G, H), jnp.float32),
        pltpu.VMEM((G, 1), jnp.float32),
    ],
)


def kernel(x, edge_index, batch, edge_attr, W1, b1, W2, b2, Wfc, bfc):
    degp = _sc_deg(edge_index, edge_attr)[:, :N].reshape(2, GRID, 1, R)
    h1 = _tc_mm(x, W1)
    hs1, dinv = _tc_scale(degp, h1)
    accp1 = _sc_agg(edge_index, edge_attr, hs1)
    h2, hs2 = _tc_mid(accp1, h1, dinv, b1.reshape(1, H), W2)
    accp2 = _sc_agg(edge_index, edge_attr, hs2)
    return _tc_final(accp2, h2, dinv, b2.reshape(1, H),
                     batch.reshape(GRID, 1, R), Wfc, bfc.reshape(1, 2))


# speedup vs baseline: 1.0028x; 1.0028x over previous
import functools

import jax
import jax.numpy as jnp
from jax import lax
from jax.experimental import pallas as pl
from jax.experimental.pallas import tpu as pltpu
from jax.experimental.pallas import tpu_sc as plsc

N = 100000
E = 3200000
D_IN = 128
H = 16
G = 64

NPAD = 100352
NTILE = NPAD // 16
ZR = 784

EPW = E // 32
SB = 800
OUTER = EPW // SB

DSB = 4000
DOUTER = EPW // DSB

R = 4000
GRID = N // R

_mesh = plsc.VectorSubcoreMesh(core_axis_name="c", subcore_axis_name="s")
_sc_params = pltpu.CompilerParams(use_tc_tiling_on_sc=False)


@functools.partial(
    pl.kernel,
    out_type=jax.ShapeDtypeStruct((2, NPAD), jnp.float32),
    mesh=_mesh,
    compiler_params=_sc_params,
    scratch_types=[
        pltpu.VMEM((2, DSB), jnp.int32),
        pltpu.VMEM((2, DSB), jnp.float32),
        pltpu.VMEM((NTILE,), jnp.float32),
        pltpu.VMEM_SHARED((NPAD,), jnp.float32),
        pltpu.SemaphoreType.DMA((2,)),
        pltpu.SemaphoreType.DMA((2,)),
    ],
)
def _sc_deg(edge_hbm, ew_hbm, out_hbm, dstb, ewb, zb, acc, esem, ssem):
    cid = lax.axis_index("c")
    sid = lax.axis_index("s")
    wid = cid * 16 + sid
    wbase = wid * EPW

    def zloop(i, _):
        zb[pl.ds(i * 16, 16)] = jnp.zeros((16,), jnp.float32)
        return 0

    lax.fori_loop(0, NTILE // 16, zloop, 0)
    pltpu.sync_copy(zb, acc.at[pl.ds(sid * NTILE, NTILE)])
    plsc.subcore_barrier()

    def stage(kc, b):
        e0 = wbase + kc * DSB
        pltpu.async_copy(edge_hbm.at[1, pl.ds(e0, DSB)], dstb.at[b],
                         esem.at[b])
        pltpu.async_copy(ew_hbm.at[pl.ds(e0, DSB)], ewb.at[b], esem.at[b])

    def unstage(kc, b):
        e0 = wbase + kc * DSB
        pltpu.make_async_copy(edge_hbm.at[1, pl.ds(e0, DSB)], dstb.at[b],
                              esem.at[b]).wait()
        pltpu.make_async_copy(ew_hbm.at[pl.ds(e0, DSB)], ewb.at[b],
                              esem.at[b]).wait()

    stage(0, 0)

    def step(kc, _):
        b = kc % 2
        unstage(kc, b)

        @pl.when(kc < DOUTER - 1)
        def _():
            @pl.when(kc >= 1)
            def _():
                pltpu.make_async_copy(ewb.at[1 - b],
                                      acc.at[dstb.at[1 - b]],
                                      ssem.at[1 - b]).wait()
            stage(kc + 1, 1 - b)

        pltpu.async_copy(ewb.at[b], acc.at[dstb.at[b]], ssem.at[b],
                         add=True)
        return 0

    lax.fori_loop(0, DOUTER, step, 0)
    pltpu.make_async_copy(ewb.at[0], acc.at[dstb.at[0]], ssem.at[0]).wait()
    pltpu.make_async_copy(ewb.at[1], acc.at[dstb.at[1]], ssem.at[1]).wait()
    plsc.subcore_barrier()
    sl = pl.ds(sid * NTILE, NTILE)
    pltpu.sync_copy(acc.at[sl], out_hbm.at[cid, sl])


@functools.partial(
    pl.kernel,
    out_type=jax.ShapeDtypeStruct((2, NPAD, H), jnp.float32),
    mesh=_mesh,
    compiler_params=_sc_params,
    scratch_types=[
        pltpu.VMEM((2, SB), jnp.int32),
        pltpu.VMEM((2, SB), jnp.int32),
        pltpu.VMEM((2, SB), jnp.float32),
        pltpu.VMEM((2, SB, H), jnp.float32),
        pltpu.VMEM_SHARED((NPAD, H), jnp.float32),
        pltpu.SemaphoreType.DMA((2,)),
        pltpu.SemaphoreType.DMA((2,)),
        pltpu.SemaphoreType.DMA((2,)),
    ],
)
def _sc_agg(edge_hbm, ew_hbm, hs_hbm, out_hbm,
            srcb, dstb, ewb, rows, acc, esem, gsem, ssem):
    cid = lax.axis_index("c")
    sid = lax.axis_index("s")
    wid = cid * 16 + sid
    wbase = wid * EPW

    def zloop(i, _):
        rows[0, i] = jnp.zeros((H,), jnp.float32)
        return 0

    lax.fori_loop(0, ZR, zloop, 0)
    base = sid * NTILE
    for t in range(NTILE // ZR):
        pltpu.sync_copy(rows.at[0, pl.ds(0, ZR)],
                        acc.at[pl.ds(base + t * ZR, ZR)])
    plsc.subcore_barrier()

    def stage(kc, b):
        e0 = wbase + kc * SB
        pltpu.async_copy(edge_hbm.at[0, pl.ds(e0, SB)], srcb.at[b],
                         esem.at[b])
        pltpu.async_copy(edge_hbm.at[1, pl.ds(e0, SB)], dstb.at[b],
                         esem.at[b])
        pltpu.async_copy(ew_hbm.at[pl.ds(e0, SB)], ewb.at[b], esem.at[b])

    def unstage(kc, b):
        e0 = wbase + kc * SB
        pltpu.make_async_copy(edge_hbm.at[0, pl.ds(e0, SB)], srcb.at[b],
                              esem.at[b]).wait()
        pltpu.make_async_copy(edge_hbm.at[1, pl.ds(e0, SB)], dstb.at[b],
                              esem.at[b]).wait()
        pltpu.make_async_copy(ew_hbm.at[pl.ds(e0, SB)], ewb.at[b],
                              esem.at[b]).wait()

    stage(0, 0)

    def step(kc, _):
        b = kc % 2
        unstage(kc, b)
        pltpu.async_copy(hs_hbm.at[srcb.at[b]], rows.at[b], gsem.at[b])

        @pl.when(kc < OUTER - 1)
        def _():
            @pl.when(kc >= 1)
            def _():
                pltpu.make_async_copy(rows.at[1 - b],
                                      acc.at[dstb.at[1 - b]],
                                      ssem.at[1 - b]).wait()
            stage(kc + 1, 1 - b)

        pltpu.make_async_copy(hs_hbm.at[srcb.at[b]], rows.at[b],
                              gsem.at[b]).wait()

        def scale(g, _):
            e0 = g * 16
            ew16 = ewb[b, pl.ds(e0, 16)]
            for l in range(16):
                rows[b, e0 + l] = rows[b, e0 + l] * ew16[l]
            return 0

        lax.fori_loop(0, SB // 16, scale, 0)
        pltpu.async_copy(rows.at[b], acc.at[dstb.at[b]], ssem.at[b],
                         add=True)
        return 0

    lax.fori_loop(0, OUTER, step, 0)
    pltpu.make_async_copy(rows.at[0], acc.at[dstb.at[0]], ssem.at[0]).wait()
    pltpu.make_async_copy(rows.at[1], acc.at[dstb.at[1]], ssem.at[1]).wait()
    plsc.subcore_barrier()
    sl = pl.ds(base, NTILE)
    pltpu.sync_copy(acc.at[sl], out_hbm.at[cid, sl])


def _pre1_body(x_ref, degp_ref, w_ref, h_ref, hs_ref, dinv_ref):
    deg = 1.0 + degp_ref[0, 0] + degp_ref[1, 0]
    dinv_row = jnp.where(deg > 0, lax.rsqrt(deg), 0.0)
    dinv = jnp.transpose(dinv_row)
    h = jnp.dot(x_ref[...], w_ref[...], preferred_element_type=jnp.float32)
    h_ref[...] = h
    hs_ref[...] = dinv * h
    dinv_ref[...] = jnp.reshape(dinv_row, (1, 1, R))


_tc_pre1 = pl.pallas_call(
    _pre1_body,
    grid=(GRID,),
    in_specs=[
        pl.BlockSpec((R, D_IN), lambda i: (i, 0)),
        pl.BlockSpec((2, 1, 1, R), lambda i: (0, i, 0, 0)),
        pl.BlockSpec((D_IN, H), lambda i: (0, 0)),
    ],
    out_specs=[
        pl.BlockSpec((R, H), lambda i: (i, 0)),
        pl.BlockSpec((R, H), lambda i: (i, 0)),
        pl.BlockSpec((1, 1, R), lambda i: (i, 0, 0)),
    ],
    out_shape=[
        jax.ShapeDtypeStruct((N, H), jnp.float32),
        jax.ShapeDtypeStruct((N, H), jnp.float32),
        jax.ShapeDtypeStruct((GRID, 1, R), jnp.float32),
    ],
)


def _mid_body(accp_ref, h1_ref, dinv_ref, b1_ref, w2_ref, h2_ref, hs2_ref):
    dinv = jnp.transpose(dinv_ref[0])
    agg = accp_ref[0] + accp_ref[1]
    z = jnp.maximum(b1_ref[...] + dinv * agg + dinv * dinv * h1_ref[...], 0.0)
    h2 = jnp.dot(z, w2_ref[...], preferred_element_type=jnp.float32)
    h2_ref[...] = h2
    hs2_ref[...] = dinv * h2


_tc_mid = pl.pallas_call(
    _mid_body,
    grid=(GRID,),
    in_specs=[
        pl.BlockSpec((2, R, H), lambda i: (0, i, 0)),
        pl.BlockSpec((R, H), lambda i: (i, 0)),
        pl.BlockSpec((1, 1, R), lambda i: (i, 0, 0)),
        pl.BlockSpec((1, H), lambda i: (0, 0)),
        pl.BlockSpec((H, H), lambda i: (0, 0)),
    ],
    out_specs=[
        pl.BlockSpec((R, H), lambda i: (i, 0)),
        pl.BlockSpec((R, H), lambda i: (i, 0)),
    ],
    out_shape=[
        jax.ShapeDtypeStruct((N, H), jnp.float32),
        jax.ShapeDtypeStruct((N, H), jnp.float32),
    ],
)


def _final_body(accp_ref, h2_ref, dinv_ref, b2_ref, batch_ref, wfc_ref,
                bfc_ref, out_ref, acc, cnt):
    i = pl.program_id(0)

    @pl.when(i == 0)
    def _():
        acc[...] = jnp.zeros((G, H), jnp.float32)
        cnt[...] = jnp.zeros((G, 1), jnp.float32)

    dinv = jnp.transpose(dinv_ref[0])
    agg = accp_ref[0] + accp_ref[1]
    z = jnp.maximum(b2_ref[...] + dinv * agg + dinv * dinv * h2_ref[...], 0.0)
    b = batch_ref[0]
    oht = (lax.broadcasted_iota(jnp.int32, (G, R), 0) == b).astype(jnp.float32)
    acc[...] += lax.dot_general(oht, z, (((1,), (0,)), ((), ())),
                                preferred_element_type=jnp.float32)
    cnt[...] += lax.dot_general(oht, jnp.ones((R, 1), jnp.float32),
                                (((1,), (0,)), ((), ())),
                                preferred_element_type=jnp.float32)

    @pl.when(i == GRID - 1)
    def _():
        pooled = acc[...] / jnp.maximum(cnt[...], 1.0)
        logits = jnp.dot(pooled, wfc_ref[...],
                         preferred_element_type=jnp.float32) + bfc_ref[...]
        m = jnp.max(logits, axis=1, keepdims=True)
        lse = m + jnp.log(jnp.sum(jnp.exp(logits - m), axis=1, keepdims=True))
        out_ref[...] = logits - lse


_tc_final = pl.pallas_call(
    _final_body,
    grid=(GRID,),
    in_specs=[
        pl.BlockSpec((2, R, H), lambda i: (0, i, 0)),
        pl.BlockSpec((R, H), lambda i: (i, 0)),
        pl.BlockSpec((1, 1, R), lambda i: (i, 0, 0)),
        pl.BlockSpec((1, H), lambda i: (0, 0)),
        pl.BlockSpec((1, 1, R), lambda i: (i, 0, 0)),
        pl.BlockSpec((H, 2), lambda i: (0, 0)),
        pl.BlockSpec((1, 2), lambda i: (0, 0)),
    ],
    out_specs=pl.BlockSpec((G, 2), lambda i: (0, 0)),
    out_shape=jax.ShapeDtypeStruct((G, 2), jnp.float32),
    scratch_shapes=[
        pltpu.VMEM((G, H), jnp.float32),
        pltpu.VMEM((G, 1), jnp.float32),
    ],
)


def kernel(x, edge_index, batch, edge_attr, W1, b1, W2, b2, Wfc, bfc):
    degp = _sc_deg(edge_index, edge_attr)[:, :N].reshape(2, GRID, 1, R)
    h1, hs1, dinv = _tc_pre1(x, degp, W1)
    accp1 = _sc_agg(edge_index, edge_attr, hs1)
    h2, hs2 = _tc_mid(accp1, h1, dinv, b1.reshape(1, H), W2)
    accp2 = _sc_agg(edge_index, edge_attr, hs2)
    return _tc_final(accp2, h2, dinv, b2.reshape(1, H),
                     batch.reshape(GRID, 1, R), Wfc, bfc.reshape(1, 2))
